# baseline (speedup 1.0000x reference)
import functools

import jax
import jax.numpy as jnp
from jax import lax
from jax.experimental import pallas as pl
from jax.experimental.pallas import tpu as pltpu
from jax.experimental.pallas import tpu_sc as plsc

N_CORES = 2
N_TILES = 16
CH = 128
SIGMA = 0.5
V = 1.0


def _spmm_sc(table, packed, wchunk, n_nodes, width, feature_split, ch, G):
    n_chunks = packed.shape[0]
    zr = 16
    n_zslabs = n_nodes // zr
    outr = 80
    n_oslabs = n_nodes // outr
    assert n_zslabs * zr == n_nodes and n_oslabs * outr == n_nodes

    per_core = n_chunks if feature_split else n_chunks // N_CORES
    cpt = per_core // N_TILES
    leftover = per_core - cpt * N_TILES
    assert cpt % G == 0
    n_groups = cpt // G

    mesh = plsc.VectorSubcoreMesh(core_axis_name="c", subcore_axis_name="s",
                                  num_cores=N_CORES, num_subcores=N_TILES)

    scratch = [
        pltpu.VMEM_SHARED((n_nodes, width), jnp.float32),
        pltpu.VMEM((2, G, 2, ch), jnp.int32),
        pltpu.VMEM((2, G, ch), jnp.float32),
        pltpu.VMEM((zr, width), jnp.float32),
        pltpu.SemaphoreType.DMA,
        pltpu.SemaphoreType.DMA,
    ]
    scratch += [pltpu.VMEM((ch, width), jnp.float32) for _ in range(G)]
    scratch += [pltpu.SemaphoreType.DMA for _ in range(G)]
    scratch += [pltpu.SemaphoreType.DMA]

    @functools.partial(
        pl.kernel,
        out_type=jax.ShapeDtypeStruct((N_CORES, n_nodes, width), jnp.float32),
        mesh=mesh,
        scratch_types=scratch,
        compiler_params=pltpu.CompilerParams(use_tc_tiling_on_sc=False),
    )
    def k(table_h, idx_h, w_h, out_h, acc_sh, ib, wb, zero_v, isem, wsem,
          *bufs):
        rows = bufs[:G]
        gsems = bufs[G:2 * G]
        ssem = bufs[2 * G]
        c = lax.axis_index("c")
        s = lax.axis_index("s")
        tbl = table_h.at[c] if feature_split else table_h
        base0 = 0 if feature_split else c * per_core

        zvec = jnp.zeros((16,), jnp.float32)

        @pl.loop(0, zr)
        def _(r):
            for jj in range(width // 16):
                zero_v[r, pl.ds(jj * 16, 16)] = zvec

        @pl.loop(s, n_zslabs, step=N_TILES)
        def _(i):
            pltpu.sync_copy(zero_v, acc_sh.at[pl.ds(i * zr, zr)])
        plsc.subcore_barrier()

        def scale_rows(rbuf, half, b):
            @pl.loop(0, ch // 16)
            def _(g):
                gbase = pl.multiple_of(g * 16, 16)
                wvec = wb[half, b, pl.ds(gbase, 16)]
                for l in range(16):
                    wl = wvec[l]
                    for jj in range(width // 16):
                        sl = pl.ds(jj * 16, 16)
                        rbuf[gbase + l, sl] = rbuf[gbase + l, sl] * wl

        tile_c0 = base0 + s * cpt

        def do_group(grp, half, prefetch_grp):
            pltpu.make_async_copy(idx_h.at[pl.ds(tile_c0, G)], ib.at[half],
                                  isem).wait()
            pltpu.make_async_copy(w_h.at[pl.ds(tile_c0, G)], wb.at[half],
                                  wsem).wait()
            gathers = [
                pltpu.async_copy(tbl.at[ib.at[half, b, 0]], rows[b], gsems[b])
                for b in range(G)
            ]
            if prefetch_grp is not None:
                @pl.when(prefetch_grp < n_groups)
                def _():
                    nc0 = tile_c0 + prefetch_grp * G
                    pltpu.async_copy(idx_h.at[pl.ds(nc0, G)],
                                     ib.at[1 - half], isem)
                    pltpu.async_copy(w_h.at[pl.ds(nc0, G)],
                                     wb.at[1 - half], wsem)
            scatters = []
            for b in range(G):
                gathers[b].wait()
                scale_rows(rows[b], half, b)
                scatters.append(
                    pltpu.async_copy(rows[b], acc_sh.at[ib.at[half, b, 1]],
                                     ssem, add=True))
            for sc in scatters:
                sc.wait()

        pltpu.async_copy(idx_h.at[pl.ds(tile_c0, G)], ib.at[0], isem)
        pltpu.async_copy(w_h.at[pl.ds(tile_c0, G)], wb.at[0], wsem)

        @pl.loop(0, n_groups // 2)
        def _(m):
            do_group(2 * m, 0, 2 * m + 1)
            do_group(2 * m + 1, 1, 2 * m + 2)

        if n_groups % 2:
            do_group(n_groups - 1, 0, None)

        if leftover:
            @pl.when(s < leftover)
            def _():
                lc = base0 + N_TILES * cpt + s
                pltpu.sync_copy(idx_h.at[pl.ds(lc, 1)], ib.at[0, pl.ds(0, 1)])
                pltpu.sync_copy(w_h.at[pl.ds(lc, 1)], wb.at[0, pl.ds(0, 1)])
                pltpu.async_copy(tbl.at[ib.at[0, 0, 0]], rows[0],
                                 gsems[0]).wait()
                scale_rows(rows[0], 0, 0)
                pltpu.sync_copy(rows[0], acc_sh.at[ib.at[0, 0, 1]], add=True)

        plsc.subcore_barrier()

        @pl.loop(s, n_oslabs, step=N_TILES)
        def _(i):
            r0 = pl.multiple_of(i * outr, outr)
            pltpu.sync_copy(acc_sh.at[pl.ds(r0, outr)],
                            out_h.at[c, pl.ds(r0, outr)])

    return k(table, packed, wchunk)


_ROWS = 1000


def _dot(a, b):
    return jnp.dot(a, b, preferred_element_type=jnp.float32)


def _ae_body(x_ref, e1w, e1b, zlw, zlb, d1w, d1b, xbw, xbb, cl,
             enc_o, z_o, xbar_o, q_o):
    xb = x_ref[...]
    e1 = jnp.maximum(_dot(xb, e1w[...]) + e1b[...], 0.0)
    z = _dot(e1, zlw[...]) + zlb[...]
    d1 = jnp.maximum(_dot(z, d1w[...]) + d1b[...], 0.0)
    xbar = _dot(d1, xbw[...]) + xbb[...]
    clv = cl[...]
    zz = jnp.sum(z * z, axis=1, keepdims=True)
    cc = jnp.sum(clv * clv, axis=1)[None, :]
    zc = lax.dot_general(z, clv, (((1,), (1,)), ((), ())),
                         preferred_element_type=jnp.float32)
    d2 = zz - 2.0 * zc + cc
    qq = 1.0 / (1.0 + d2 / V)
    q = qq / jnp.sum(qq, axis=1, keepdims=True)
    enc_o[...] = e1
    z_o[...] = z
    xbar_o[...] = xbar
    q_o[...] = q


def _dense_ae(x, enc1_w, enc1_b, zl_w, zl_b, dec1_w, dec1_b,
              xbar_w, xbar_b, cluster):
    n, n_in = x.shape
    n_e1 = enc1_w.shape[1]
    n_z = zl_w.shape[1]
    n_k = cluster.shape[0]
    grid = n // _ROWS
    full = lambda shp: pl.BlockSpec(shp, lambda i: (0,) * len(shp))
    row = lambda d: pl.BlockSpec((_ROWS, d), lambda i: (i, 0))
    return pl.pallas_call(
        _ae_body,
        grid=(grid,),
        in_specs=[
            row(n_in),
            full(enc1_w.shape), full((1, n_e1)),
            full(zl_w.shape), full((1, n_z)),
            full(dec1_w.shape), full((1, n_e1)),
            full(xbar_w.shape), full((1, n_in)),
            full(cluster.shape),
        ],
        out_specs=[row(n_e1), row(n_z), row(n_in), row(n_k)],
        out_shape=[
            jax.ShapeDtypeStruct((n, n_e1), jnp.float32),
            jax.ShapeDtypeStruct((n, n_z), jnp.float32),
            jax.ShapeDtypeStruct((n, n_in), jnp.float32),
            jax.ShapeDtypeStruct((n, n_k), jnp.float32),
        ],
    )(x, enc1_w, enc1_b.reshape(1, -1), zl_w, zl_b.reshape(1, -1),
      dec1_w, dec1_b.reshape(1, -1), xbar_w, xbar_b.reshape(1, -1), cluster)


def _gcn12_body(agg_ref, enc_ref, g1_ref, g4_ref, s2_o):
    h1 = jnp.maximum(_dot(agg_ref[0], g1_ref[0]) + _dot(agg_ref[1], g1_ref[1]),
                     0.0)
    mix = (1.0 - SIGMA) * h1 + SIGMA * enc_ref[...]
    s2_o[...] = _dot(mix, g4_ref[...])


def _gcn12(agg1, enc_h1, g1r, g4_w):
    n = enc_h1.shape[0]
    n_e1 = enc_h1.shape[1]
    n_z = g4_w.shape[1]
    hw = agg1.shape[2]
    grid = n // _ROWS
    return pl.pallas_call(
        _gcn12_body,
        grid=(grid,),
        in_specs=[
            pl.BlockSpec((N_CORES, _ROWS, hw), lambda i: (0, i, 0)),
            pl.BlockSpec((_ROWS, n_e1), lambda i: (i, 0)),
            pl.BlockSpec(g1r.shape, lambda i: (0, 0, 0)),
            pl.BlockSpec(g4_w.shape, lambda i: (0, 0)),
        ],
        out_specs=pl.BlockSpec((_ROWS, n_z), lambda i: (i, 0)),
        out_shape=jax.ShapeDtypeStruct((n, n_z), jnp.float32),
    )(agg1, enc_h1, g1r, g4_w)


def _gcn3_body(agg_ref, z_ref, g5_ref, s3_o):
    h2 = jnp.maximum(agg_ref[0] + agg_ref[1], 0.0)
    mix = (1.0 - SIGMA) * h2 + SIGMA * z_ref[...]
    s3_o[...] = _dot(mix, g5_ref[...])


def _gcn3(agg2, z, g5_w):
    n, n_z = z.shape
    n_k = g5_w.shape[1]
    grid = n // _ROWS
    return pl.pallas_call(
        _gcn3_body,
        grid=(grid,),
        in_specs=[
            pl.BlockSpec((N_CORES, _ROWS, n_z), lambda i: (0, i, 0)),
            pl.BlockSpec((_ROWS, n_z), lambda i: (i, 0)),
            pl.BlockSpec(g5_w.shape, lambda i: (0, 0)),
        ],
        out_specs=pl.BlockSpec((_ROWS, n_k), lambda i: (i, 0)),
        out_shape=jax.ShapeDtypeStruct((n, n_k), jnp.float32),
    )(agg2, z, g5_w)


def _softmax_body(agg_ref, pred_o):
    h3 = agg_ref[0] + agg_ref[1]
    m = jnp.max(h3, axis=1, keepdims=True)
    e = jnp.exp(h3 - m)
    pred_o[...] = e / jnp.sum(e, axis=1, keepdims=True)


def _softmax(agg3):
    n_k = agg3.shape[2]
    n = agg3.shape[1]
    grid = n // _ROWS
    return pl.pallas_call(
        _softmax_body,
        grid=(grid,),
        in_specs=[pl.BlockSpec((N_CORES, _ROWS, n_k), lambda i: (0, i, 0))],
        out_specs=pl.BlockSpec((_ROWS, n_k), lambda i: (i, 0)),
        out_shape=jax.ShapeDtypeStruct((n, n_k), jnp.float32),
    )(agg3)


def kernel(x, edge_index, edge_weight, enc1_w, enc1_b, zl_w, zl_b,
           dec1_w, dec1_b, xbar_w, xbar_b, g1_w, g4_w, g5_w, cluster):
    n, n_in = x.shape
    hw = n_in // N_CORES
    src = edge_index[0]
    dst = edge_index[1]
    def pack_idx(ch):
        e = src.shape[0]
        return jnp.stack([src.reshape(e // ch, ch),
                          dst.reshape(e // ch, ch)], axis=1)

    pk64 = pack_idx(64)
    pk128 = pack_idx(128)
    w64 = edge_weight.reshape(-1, 64)
    w128 = edge_weight.reshape(-1, 128)

    xs = jnp.stack([x[:, :hw], x[:, hw:]])
    g1r = g1_w.reshape(N_CORES, hw, g1_w.shape[1])

    agg1 = _spmm_sc(xs, pk64, w64, n, hw, feature_split=True, ch=64, G=4)

    enc_h1, z, x_bar, q = _dense_ae(
        x, enc1_w, enc1_b, zl_w, zl_b, dec1_w, dec1_b, xbar_w, xbar_b,
        cluster)

    s2 = _gcn12(agg1, enc_h1, g1r, g4_w)

    agg2 = _spmm_sc(s2, pk128, w128, n, s2.shape[1], feature_split=False,
                    ch=128, G=3)

    s3 = _gcn3(agg2, z, g5_w)

    agg3 = _spmm_sc(s3, pk128, w128, n, s3.shape[1], feature_split=False,
                    ch=128, G=13)

    predict = _softmax(agg3)

    return (x_bar, q, predict, z)


# speedup vs baseline: 1.1407x; 1.1407x over previous
import functools

import jax
import jax.numpy as jnp
from jax import lax
from jax.experimental import pallas as pl
from jax.experimental.pallas import tpu as pltpu
from jax.experimental.pallas import tpu_sc as plsc

N_CORES = 2
N_TILES = 16
CH = 128
SIGMA = 0.5
V = 1.0


def _spmm_sc(table, packed, wchunk, n_nodes, width, feature_split, ch, G):
    n_chunks = packed.shape[0]
    zr = 16
    n_zslabs = n_nodes // zr
    outr = 80
    n_oslabs = n_nodes // outr
    assert n_zslabs * zr == n_nodes and n_oslabs * outr == n_nodes

    per_core = n_chunks if feature_split else n_chunks // N_CORES
    cpt = per_core // N_TILES
    leftover = per_core - cpt * N_TILES
    assert cpt % G == 0
    n_groups = cpt // G

    mesh = plsc.VectorSubcoreMesh(core_axis_name="c", subcore_axis_name="s",
                                  num_cores=N_CORES, num_subcores=N_TILES)

    scratch = [
        pltpu.VMEM_SHARED((n_nodes, width), jnp.float32),
        pltpu.VMEM((2, G, 2, ch), jnp.int32),
        pltpu.VMEM((2, G, ch), jnp.float32),
        pltpu.VMEM((zr, width), jnp.float32),
        pltpu.SemaphoreType.DMA,
        pltpu.SemaphoreType.DMA,
    ]
    scratch += [pltpu.VMEM((ch, width), jnp.float32) for _ in range(G)]
    scratch += [pltpu.SemaphoreType.DMA for _ in range(G)]
    scratch += [pltpu.SemaphoreType.DMA]

    @functools.partial(
        pl.kernel,
        out_type=jax.ShapeDtypeStruct((N_CORES, n_nodes, width), jnp.float32),
        mesh=mesh,
        scratch_types=scratch,
        compiler_params=pltpu.CompilerParams(use_tc_tiling_on_sc=False),
    )
    def k(table_h, idx_h, w_h, out_h, acc_sh, ib, wb, zero_v, isem, wsem,
          *bufs):
        rows = bufs[:G]
        gsems = bufs[G:2 * G]
        ssem = bufs[2 * G]
        c = lax.axis_index("c")
        s = lax.axis_index("s")
        tbl = table_h.at[c] if feature_split else table_h
        base0 = 0 if feature_split else c * per_core

        zvec = jnp.zeros((16,), jnp.float32)

        @pl.loop(0, zr)
        def _(r):
            for jj in range(width // 16):
                zero_v[r, pl.ds(jj * 16, 16)] = zvec

        @pl.loop(s, n_zslabs, step=N_TILES)
        def _(i):
            pltpu.sync_copy(zero_v, acc_sh.at[pl.ds(i * zr, zr)])
        plsc.subcore_barrier()

        def scale_rows(rbuf, half, b):
            @pl.loop(0, ch // 16)
            def _(g):
                gbase = pl.multiple_of(g * 16, 16)
                wvec = wb[half, b, pl.ds(gbase, 16)]
                for l in range(16):
                    wl = wvec[l]
                    for jj in range(width // 16):
                        sl = pl.ds(jj * 16, 16)
                        rbuf[gbase + l, sl] = rbuf[gbase + l, sl] * wl

        tile_c0 = base0 + s * cpt

        def do_group(grp, half, prefetch_grp):
            pltpu.make_async_copy(idx_h.at[pl.ds(tile_c0, G)], ib.at[half],
                                  isem).wait()
            pltpu.make_async_copy(w_h.at[pl.ds(tile_c0, G)], wb.at[half],
                                  wsem).wait()
            gathers = [
                pltpu.async_copy(tbl.at[ib.at[half, b, 0]], rows[b], gsems[b])
                for b in range(G)
            ]
            if prefetch_grp is not None:
                @pl.when(prefetch_grp < n_groups)
                def _():
                    nc0 = tile_c0 + prefetch_grp * G
                    pltpu.async_copy(idx_h.at[pl.ds(nc0, G)],
                                     ib.at[1 - half], isem)
                    pltpu.async_copy(w_h.at[pl.ds(nc0, G)],
                                     wb.at[1 - half], wsem)
            scatters = []
            for b in range(G):
                gathers[b].wait()
                scale_rows(rows[b], half, b)
                scatters.append(
                    pltpu.async_copy(rows[b], acc_sh.at[ib.at[half, b, 1]],
                                     ssem, add=True))
            for sc in scatters:
                sc.wait()

        pltpu.async_copy(idx_h.at[pl.ds(tile_c0, G)], ib.at[0], isem)
        pltpu.async_copy(w_h.at[pl.ds(tile_c0, G)], wb.at[0], wsem)

        @pl.loop(0, n_groups // 2)
        def _(m):
            do_group(2 * m, 0, 2 * m + 1)
            do_group(2 * m + 1, 1, 2 * m + 2)

        if n_groups % 2:
            do_group(n_groups - 1, 0, None)

        if leftover:
            @pl.when(s < leftover)
            def _():
                lc = base0 + N_TILES * cpt + s
                pltpu.sync_copy(idx_h.at[pl.ds(lc, 1)], ib.at[0, pl.ds(0, 1)])
                pltpu.sync_copy(w_h.at[pl.ds(lc, 1)], wb.at[0, pl.ds(0, 1)])
                pltpu.async_copy(tbl.at[ib.at[0, 0, 0]], rows[0],
                                 gsems[0]).wait()
                scale_rows(rows[0], 0, 0)
                pltpu.sync_copy(rows[0], acc_sh.at[ib.at[0, 0, 1]], add=True)

        plsc.subcore_barrier()

        @pl.loop(s, n_oslabs, step=N_TILES)
        def _(i):
            r0 = pl.multiple_of(i * outr, outr)
            pltpu.sync_copy(acc_sh.at[pl.ds(r0, outr)],
                            out_h.at[c, pl.ds(r0, outr)])

    return k(table, packed, wchunk)


_ROWS = 1000


def _dot(a, b):
    return jnp.dot(a, b, preferred_element_type=jnp.float32)


def _ae_body(x_ref, e1w, e1b, zlw, zlb, d1w, d1b, xbw, xbb, cl,
             enc_o, z_o, xbar_o, q_o):
    xb = x_ref[...]
    e1 = jnp.maximum(_dot(xb, e1w[...]) + e1b[...], 0.0)
    z = _dot(e1, zlw[...]) + zlb[...]
    d1 = jnp.maximum(_dot(z, d1w[...]) + d1b[...], 0.0)
    xbar = _dot(d1, xbw[...]) + xbb[...]
    clv = cl[...]
    zz = jnp.sum(z * z, axis=1, keepdims=True)
    cc = jnp.sum(clv * clv, axis=1)[None, :]
    zc = lax.dot_general(z, clv, (((1,), (1,)), ((), ())),
                         preferred_element_type=jnp.float32)
    d2 = zz - 2.0 * zc + cc
    qq = 1.0 / (1.0 + d2 / V)
    q = qq / jnp.sum(qq, axis=1, keepdims=True)
    enc_o[...] = e1
    z_o[...] = z
    xbar_o[...] = xbar
    q_o[...] = q


def _dense_ae(x, enc1_w, enc1_b, zl_w, zl_b, dec1_w, dec1_b,
              xbar_w, xbar_b, cluster):
    n, n_in = x.shape
    n_e1 = enc1_w.shape[1]
    n_z = zl_w.shape[1]
    n_k = cluster.shape[0]
    grid = n // _ROWS
    full = lambda shp: pl.BlockSpec(shp, lambda i: (0,) * len(shp))
    row = lambda d: pl.BlockSpec((_ROWS, d), lambda i: (i, 0))
    return pl.pallas_call(
        _ae_body,
        grid=(grid,),
        in_specs=[
            row(n_in),
            full(enc1_w.shape), full((1, n_e1)),
            full(zl_w.shape), full((1, n_z)),
            full(dec1_w.shape), full((1, n_e1)),
            full(xbar_w.shape), full((1, n_in)),
            full(cluster.shape),
        ],
        out_specs=[row(n_e1), row(n_z), row(n_in), row(n_k)],
        out_shape=[
            jax.ShapeDtypeStruct((n, n_e1), jnp.float32),
            jax.ShapeDtypeStruct((n, n_z), jnp.float32),
            jax.ShapeDtypeStruct((n, n_in), jnp.float32),
            jax.ShapeDtypeStruct((n, n_k), jnp.float32),
        ],
    )(x, enc1_w, enc1_b.reshape(1, -1), zl_w, zl_b.reshape(1, -1),
      dec1_w, dec1_b.reshape(1, -1), xbar_w, xbar_b.reshape(1, -1), cluster)


def _gcn12_body(agg_ref, enc_ref, g1_ref, g4_ref, s2_o):
    h1 = jnp.maximum(_dot(agg_ref[0], g1_ref[0]) + _dot(agg_ref[1], g1_ref[1]),
                     0.0)
    mix = (1.0 - SIGMA) * h1 + SIGMA * enc_ref[...]
    s2 = _dot(mix, g4_ref[...])
    hz = s2.shape[1] // 2
    s2_o[0] = s2[:, :hz]
    s2_o[1] = s2[:, hz:]


def _gcn12(agg1, enc_h1, g1r, g4_w):
    n = enc_h1.shape[0]
    n_e1 = enc_h1.shape[1]
    n_z = g4_w.shape[1]
    hw = agg1.shape[2]
    grid = n // _ROWS
    return pl.pallas_call(
        _gcn12_body,
        grid=(grid,),
        in_specs=[
            pl.BlockSpec((N_CORES, _ROWS, hw), lambda i: (0, i, 0)),
            pl.BlockSpec((_ROWS, n_e1), lambda i: (i, 0)),
            pl.BlockSpec(g1r.shape, lambda i: (0, 0, 0)),
            pl.BlockSpec(g4_w.shape, lambda i: (0, 0)),
        ],
        out_specs=pl.BlockSpec((N_CORES, _ROWS, n_z // 2),
                               lambda i: (0, i, 0)),
        out_shape=jax.ShapeDtypeStruct((N_CORES, n, n_z // 2), jnp.float32),
    )(agg1, enc_h1, g1r, g4_w)


def _gcn3_body(agg_ref, z_ref, g5_ref, s3_o):
    h2 = jnp.maximum(jnp.concatenate([agg_ref[0], agg_ref[1]], axis=1), 0.0)
    mix = (1.0 - SIGMA) * h2 + SIGMA * z_ref[...]
    s3_o[...] = _dot(mix, g5_ref[...])


def _gcn3(agg2, z, g5_w):
    n, n_z = z.shape
    n_k = g5_w.shape[1]
    aw = agg2.shape[2]
    grid = n // _ROWS
    return pl.pallas_call(
        _gcn3_body,
        grid=(grid,),
        in_specs=[
            pl.BlockSpec((N_CORES, _ROWS, aw), lambda i: (0, i, 0)),
            pl.BlockSpec((_ROWS, n_z), lambda i: (i, 0)),
            pl.BlockSpec(g5_w.shape, lambda i: (0, 0)),
        ],
        out_specs=pl.BlockSpec((_ROWS, n_k), lambda i: (i, 0)),
        out_shape=jax.ShapeDtypeStruct((n, n_k), jnp.float32),
    )(agg2, z, g5_w)


def _softmax_body(agg_ref, pred_o):
    h3 = agg_ref[0] + agg_ref[1]
    m = jnp.max(h3, axis=1, keepdims=True)
    e = jnp.exp(h3 - m)
    pred_o[...] = e / jnp.sum(e, axis=1, keepdims=True)


def _softmax(agg3):
    n_k = agg3.shape[2]
    n = agg3.shape[1]
    grid = n // _ROWS
    return pl.pallas_call(
        _softmax_body,
        grid=(grid,),
        in_specs=[pl.BlockSpec((N_CORES, _ROWS, n_k), lambda i: (0, i, 0))],
        out_specs=pl.BlockSpec((_ROWS, n_k), lambda i: (i, 0)),
        out_shape=jax.ShapeDtypeStruct((n, n_k), jnp.float32),
    )(agg3)


def kernel(x, edge_index, edge_weight, enc1_w, enc1_b, zl_w, zl_b,
           dec1_w, dec1_b, xbar_w, xbar_b, g1_w, g4_w, g5_w, cluster):
    n, n_in = x.shape
    hw = n_in // N_CORES
    src = edge_index[0]
    dst = edge_index[1]
    def pack_idx(ch):
        e = src.shape[0]
        return jnp.stack([src.reshape(e // ch, ch),
                          dst.reshape(e // ch, ch)], axis=1)

    pk64 = pack_idx(64)
    pk128 = pack_idx(128)
    w64 = edge_weight.reshape(-1, 64)
    w128 = edge_weight.reshape(-1, 128)

    xs = jnp.stack([x[:, :hw], x[:, hw:]])
    g1r = g1_w.reshape(N_CORES, hw, g1_w.shape[1])

    agg1 = _spmm_sc(xs, pk64, w64, n, hw, feature_split=True, ch=64, G=4)

    enc_h1, z, x_bar, q = _dense_ae(
        x, enc1_w, enc1_b, zl_w, zl_b, dec1_w, dec1_b, xbar_w, xbar_b,
        cluster)

    s2 = _gcn12(agg1, enc_h1, g1r, g4_w)

    agg2 = _spmm_sc(s2, pk128, w128, n, s2.shape[2], feature_split=True,
                    ch=128, G=13)

    s3 = _gcn3(agg2, z, g5_w)

    agg3 = _spmm_sc(s3, pk128, w128, n, s3.shape[1], feature_split=False,
                    ch=128, G=13)

    predict = _softmax(agg3)

    return (x_bar, q, predict, z)


# speedup vs baseline: 1.1729x; 1.0282x over previous
import functools

import jax
import jax.numpy as jnp
from jax import lax
from jax.experimental import pallas as pl
from jax.experimental.pallas import tpu as pltpu
from jax.experimental.pallas import tpu_sc as plsc

N_CORES = 2
N_TILES = 16
CH = 128
SIGMA = 0.5
V = 1.0


def _spmm_sc(table, packed, wchunk, n_nodes, width, feature_split, ch, G):
    n_chunks = packed.shape[0]
    zr = 16
    n_zslabs = n_nodes // zr
    outr = 80
    n_oslabs = n_nodes // outr
    assert n_zslabs * zr == n_nodes and n_oslabs * outr == n_nodes

    per_core = n_chunks if feature_split else n_chunks // N_CORES
    cpt = per_core // N_TILES
    leftover = per_core - cpt * N_TILES
    assert cpt % G == 0
    n_groups = cpt // G

    mesh = plsc.VectorSubcoreMesh(core_axis_name="c", subcore_axis_name="s",
                                  num_cores=N_CORES, num_subcores=N_TILES)

    bf16 = table.dtype == jnp.bfloat16

    scratch = [
        pltpu.VMEM_SHARED((n_nodes, width), jnp.float32),
        pltpu.VMEM((2, G, 2, ch), jnp.int32),
        pltpu.VMEM((2, G, ch), jnp.float32),
        pltpu.VMEM((zr, width), jnp.float32),
        pltpu.SemaphoreType.DMA,
        pltpu.SemaphoreType.DMA,
        pltpu.SemaphoreType.DMA,
    ]
    scratch += [pltpu.VMEM((ch, width), jnp.float32) for _ in range(G)]
    if bf16:
        scratch += [pltpu.VMEM((ch, width), jnp.bfloat16) for _ in range(G)]
    scratch += [pltpu.SemaphoreType.DMA for _ in range(G)]
    scratch += [pltpu.SemaphoreType.DMA]

    @functools.partial(
        pl.kernel,
        out_type=jax.ShapeDtypeStruct((N_CORES, n_nodes, width), jnp.float32),
        mesh=mesh,
        scratch_types=scratch,
        compiler_params=pltpu.CompilerParams(use_tc_tiling_on_sc=False),
    )
    def k(table_h, idx_h, w_h, out_h, acc_sh, ib, wb, zero_v, isem, wsem,
          zsem, *bufs):
        rows = bufs[:G]
        nb = 2 * G if bf16 else G
        rows16 = bufs[G:nb] if bf16 else rows
        gsems = bufs[nb:nb + G]
        ssem = bufs[nb + G]
        c = lax.axis_index("c")
        s = lax.axis_index("s")
        tbl = table_h.at[c] if feature_split else table_h
        base0 = 0 if feature_split else c * per_core

        zvec = jnp.zeros((16,), jnp.float32)

        @pl.loop(0, zr)
        def _(r):
            for jj in range(width // 16):
                zero_v[r, pl.ds(jj * 16, 16)] = zvec

        @pl.loop(s, n_zslabs, step=N_TILES)
        def _(i):
            pltpu.async_copy(zero_v, acc_sh.at[pl.ds(i * zr, zr)], zsem)

        @pl.loop(s, n_zslabs, step=N_TILES)
        def _(i):
            pltpu.make_async_copy(
                zero_v, acc_sh.at[pl.ds(s * zr, zr)], zsem).wait()
        plsc.subcore_barrier()

        def scale_rows(half, b):
            @pl.loop(0, ch // 16)
            def _(g):
                gbase = pl.multiple_of(g * 16, 16)
                wvec = wb[half, b, pl.ds(gbase, 16)]
                for l in range(16):
                    wl = wvec[l]
                    if bf16:
                        for q in range(width // 32):
                            v = rows16[b][gbase + l, pl.ds(q * 32, 32)]
                            a, d = plsc.unpack(
                                v, format=plsc.PackFormat.INTERLEAVED)
                            rows[b][gbase + l, pl.ds(q * 32, 16)] = a * wl
                            rows[b][gbase + l, pl.ds(q * 32 + 16, 16)] = d * wl
                    else:
                        for jj in range(width // 16):
                            sl = pl.ds(jj * 16, 16)
                            rows[b][gbase + l, sl] = rows[b][gbase + l, sl] * wl

        tile_c0 = base0 + s * cpt

        def do_group(grp, half, prefetch_grp):
            pltpu.make_async_copy(idx_h.at[pl.ds(tile_c0, G)], ib.at[half],
                                  isem).wait()
            pltpu.make_async_copy(w_h.at[pl.ds(tile_c0, G)], wb.at[half],
                                  wsem).wait()
            gathers = [
                pltpu.async_copy(tbl.at[ib.at[half, b, 0]], rows16[b],
                                 gsems[b])
                for b in range(G)
            ]
            if prefetch_grp is not None:
                @pl.when(prefetch_grp < n_groups)
                def _():
                    nc0 = tile_c0 + prefetch_grp * G
                    pltpu.async_copy(idx_h.at[pl.ds(nc0, G)],
                                     ib.at[1 - half], isem)
                    pltpu.async_copy(w_h.at[pl.ds(nc0, G)],
                                     wb.at[1 - half], wsem)
            scatters = []
            for b in range(G):
                gathers[b].wait()
                scale_rows(half, b)
                scatters.append(
                    pltpu.async_copy(rows[b], acc_sh.at[ib.at[half, b, 1]],
                                     ssem, add=True))
            for sc in scatters:
                sc.wait()

        pltpu.async_copy(idx_h.at[pl.ds(tile_c0, G)], ib.at[0], isem)
        pltpu.async_copy(w_h.at[pl.ds(tile_c0, G)], wb.at[0], wsem)

        @pl.loop(0, n_groups // 2)
        def _(m):
            do_group(2 * m, 0, 2 * m + 1)
            do_group(2 * m + 1, 1, 2 * m + 2)

        if n_groups % 2:
            do_group(n_groups - 1, 0, None)

        if leftover:
            @pl.when(s < leftover)
            def _():
                lc = base0 + N_TILES * cpt + s
                pltpu.sync_copy(idx_h.at[pl.ds(lc, 1)], ib.at[0, pl.ds(0, 1)])
                pltpu.sync_copy(w_h.at[pl.ds(lc, 1)], wb.at[0, pl.ds(0, 1)])
                pltpu.async_copy(tbl.at[ib.at[0, 0, 0]], rows16[0],
                                 gsems[0]).wait()
                scale_rows(0, 0)
                pltpu.sync_copy(rows[0], acc_sh.at[ib.at[0, 0, 1]], add=True)

        plsc.subcore_barrier()

        @pl.loop(s, n_oslabs, step=N_TILES)
        def _(i):
            r0 = pl.multiple_of(i * outr, outr)
            pltpu.async_copy(acc_sh.at[pl.ds(r0, outr)],
                             out_h.at[c, pl.ds(r0, outr)], zsem)

        @pl.loop(s, n_oslabs, step=N_TILES)
        def _(i):
            r0 = pl.multiple_of(s * outr, outr)
            pltpu.make_async_copy(acc_sh.at[pl.ds(r0, outr)],
                                  out_h.at[c, pl.ds(r0, outr)], zsem).wait()

    return k(table, packed, wchunk)


_ROWS = 1000


def _dot(a, b):
    return jnp.dot(a, b, preferred_element_type=jnp.float32)


def _ae_body(x_ref, e1w, e1b, zlw, zlb, d1w, d1b, xbw, xbb, cl,
             enc_o, z_o, xbar_o, q_o):
    xb = x_ref[...]
    e1 = jnp.maximum(_dot(xb, e1w[...]) + e1b[...], 0.0)
    z = _dot(e1, zlw[...]) + zlb[...]
    d1 = jnp.maximum(_dot(z, d1w[...]) + d1b[...], 0.0)
    xbar = _dot(d1, xbw[...]) + xbb[...]
    clv = cl[...]
    zz = jnp.sum(z * z, axis=1, keepdims=True)
    cc = jnp.sum(clv * clv, axis=1)[None, :]
    zc = lax.dot_general(z, clv, (((1,), (1,)), ((), ())),
                         preferred_element_type=jnp.float32)
    d2 = zz - 2.0 * zc + cc
    qq = 1.0 / (1.0 + d2 / V)
    q = qq / jnp.sum(qq, axis=1, keepdims=True)
    enc_o[...] = e1
    z_o[...] = z
    xbar_o[...] = xbar
    q_o[...] = q


def _dense_ae(x, enc1_w, enc1_b, zl_w, zl_b, dec1_w, dec1_b,
              xbar_w, xbar_b, cluster):
    n, n_in = x.shape
    n_e1 = enc1_w.shape[1]
    n_z = zl_w.shape[1]
    n_k = cluster.shape[0]
    grid = n // _ROWS
    full = lambda shp: pl.BlockSpec(shp, lambda i: (0,) * len(shp))
    row = lambda d: pl.BlockSpec((_ROWS, d), lambda i: (i, 0))
    return pl.pallas_call(
        _ae_body,
        grid=(grid,),
        in_specs=[
            row(n_in),
            full(enc1_w.shape), full((1, n_e1)),
            full(zl_w.shape), full((1, n_z)),
            full(dec1_w.shape), full((1, n_e1)),
            full(xbar_w.shape), full((1, n_in)),
            full(cluster.shape),
        ],
        out_specs=[row(n_e1), row(n_z), row(n_in), row(n_k)],
        out_shape=[
            jax.ShapeDtypeStruct((n, n_e1), jnp.float32),
            jax.ShapeDtypeStruct((n, n_z), jnp.float32),
            jax.ShapeDtypeStruct((n, n_in), jnp.float32),
            jax.ShapeDtypeStruct((n, n_k), jnp.float32),
        ],
    )(x, enc1_w, enc1_b.reshape(1, -1), zl_w, zl_b.reshape(1, -1),
      dec1_w, dec1_b.reshape(1, -1), xbar_w, xbar_b.reshape(1, -1), cluster)


def _gcn12_body(agg_ref, enc_ref, g1_ref, g4_ref, s2_o):
    h1 = jnp.maximum(_dot(agg_ref[0], g1_ref[0]) + _dot(agg_ref[1], g1_ref[1]),
                     0.0)
    mix = (1.0 - SIGMA) * h1 + SIGMA * enc_ref[...]
    s2 = _dot(mix, g4_ref[...])
    hz = s2.shape[1] // 2
    s2_o[0] = s2[:, :hz]
    s2_o[1] = s2[:, hz:]


def _gcn12(agg1, enc_h1, g1r, g4_w):
    n = enc_h1.shape[0]
    n_e1 = enc_h1.shape[1]
    n_z = g4_w.shape[1]
    hw = agg1.shape[2]
    grid = n // _ROWS
    return pl.pallas_call(
        _gcn12_body,
        grid=(grid,),
        in_specs=[
            pl.BlockSpec((N_CORES, _ROWS, hw), lambda i: (0, i, 0)),
            pl.BlockSpec((_ROWS, n_e1), lambda i: (i, 0)),
            pl.BlockSpec(g1r.shape, lambda i: (0, 0, 0)),
            pl.BlockSpec(g4_w.shape, lambda i: (0, 0)),
        ],
        out_specs=pl.BlockSpec((N_CORES, _ROWS, n_z // 2),
                               lambda i: (0, i, 0)),
        out_shape=jax.ShapeDtypeStruct((N_CORES, n, n_z // 2), jnp.float32),
    )(agg1, enc_h1, g1r, g4_w)


def _gcn3_body(agg_ref, z_ref, g5_ref, s3_o):
    h2 = jnp.maximum(jnp.concatenate([agg_ref[0], agg_ref[1]], axis=1), 0.0)
    mix = (1.0 - SIGMA) * h2 + SIGMA * z_ref[...]
    s3_o[...] = _dot(mix, g5_ref[...])


def _gcn3(agg2, z, g5_w):
    n, n_z = z.shape
    n_k = g5_w.shape[1]
    aw = agg2.shape[2]
    grid = n // _ROWS
    return pl.pallas_call(
        _gcn3_body,
        grid=(grid,),
        in_specs=[
            pl.BlockSpec((N_CORES, _ROWS, aw), lambda i: (0, i, 0)),
            pl.BlockSpec((_ROWS, n_z), lambda i: (i, 0)),
            pl.BlockSpec(g5_w.shape, lambda i: (0, 0)),
        ],
        out_specs=pl.BlockSpec((_ROWS, n_k), lambda i: (i, 0)),
        out_shape=jax.ShapeDtypeStruct((n, n_k), jnp.float32),
    )(agg2, z, g5_w)


def _softmax_body(agg_ref, pred_o):
    h3 = agg_ref[0] + agg_ref[1]
    m = jnp.max(h3, axis=1, keepdims=True)
    e = jnp.exp(h3 - m)
    pred_o[...] = e / jnp.sum(e, axis=1, keepdims=True)


def _softmax(agg3):
    n_k = agg3.shape[2]
    n = agg3.shape[1]
    grid = n // _ROWS
    return pl.pallas_call(
        _softmax_body,
        grid=(grid,),
        in_specs=[pl.BlockSpec((N_CORES, _ROWS, n_k), lambda i: (0, i, 0))],
        out_specs=pl.BlockSpec((_ROWS, n_k), lambda i: (i, 0)),
        out_shape=jax.ShapeDtypeStruct((n, n_k), jnp.float32),
    )(agg3)


def kernel(x, edge_index, edge_weight, enc1_w, enc1_b, zl_w, zl_b,
           dec1_w, dec1_b, xbar_w, xbar_b, g1_w, g4_w, g5_w, cluster):
    n, n_in = x.shape
    hw = n_in // N_CORES
    src = edge_index[0]
    dst = edge_index[1]
    def pack_idx(ch):
        e = src.shape[0]
        return jnp.stack([src.reshape(e // ch, ch),
                          dst.reshape(e // ch, ch)], axis=1)

    pk64 = pack_idx(64)
    pk128 = pack_idx(128)
    w64 = edge_weight.reshape(-1, 64)
    w128 = edge_weight.reshape(-1, 128)

    xs = jnp.stack([x[:, :hw], x[:, hw:]])
    g1r = g1_w.reshape(N_CORES, hw, g1_w.shape[1])

    agg1 = _spmm_sc(xs, pk64, w64, n, hw, feature_split=True, ch=64, G=4)

    enc_h1, z, x_bar, q = _dense_ae(
        x, enc1_w, enc1_b, zl_w, zl_b, dec1_w, dec1_b, xbar_w, xbar_b,
        cluster)

    s2 = _gcn12(agg1, enc_h1, g1r, g4_w)

    agg2 = _spmm_sc(s2, pk128, w128, n, s2.shape[2], feature_split=True,
                    ch=128, G=13)

    s3 = _gcn3(agg2, z, g5_w)

    agg3 = _spmm_sc(s3, pk128, w128, n, s3.shape[1], feature_split=False,
                    ch=128, G=13)

    predict = _softmax(agg3)

    return (x_bar, q, predict, z)


# speedup vs baseline: 1.2527x; 1.0680x over previous
import functools

import jax
import jax.numpy as jnp
from jax import lax
from jax.experimental import pallas as pl
from jax.experimental.pallas import tpu as pltpu
from jax.experimental.pallas import tpu_sc as plsc

N_CORES = 2
N_TILES = 16
CH = 128
SIGMA = 0.5
V = 1.0


def _spmm_sc(table, packed, wchunk, n_nodes, width, feature_split, ch, G):
    n_chunks = packed.shape[0]
    zr = 16
    n_zslabs = n_nodes // zr
    outr = 80
    n_oslabs = n_nodes // outr
    assert n_zslabs * zr == n_nodes and n_oslabs * outr == n_nodes

    per_core = n_chunks if feature_split else n_chunks // N_CORES
    cpt = per_core // N_TILES
    leftover = per_core - cpt * N_TILES
    assert cpt % G == 0
    n_groups = cpt // G

    mesh = plsc.VectorSubcoreMesh(core_axis_name="c", subcore_axis_name="s",
                                  num_cores=N_CORES, num_subcores=N_TILES)

    bf16 = table.dtype == jnp.bfloat16

    scratch = [
        pltpu.VMEM_SHARED((n_nodes, width), jnp.float32),
        pltpu.VMEM((2, G, 2, ch), jnp.int32),
        pltpu.VMEM((2, G, ch), jnp.float32),
        pltpu.VMEM((zr, width), jnp.float32),
        pltpu.SemaphoreType.DMA,
        pltpu.SemaphoreType.DMA,
        pltpu.SemaphoreType.DMA,
    ]
    scratch += [pltpu.VMEM((ch, width), jnp.float32) for _ in range(G)]
    if bf16:
        scratch += [pltpu.VMEM((ch, width), jnp.bfloat16) for _ in range(G)]
    scratch += [pltpu.SemaphoreType.DMA for _ in range(G)]
    scratch += [pltpu.SemaphoreType.DMA]

    @functools.partial(
        pl.kernel,
        out_type=jax.ShapeDtypeStruct((N_CORES, n_nodes, width), jnp.float32),
        mesh=mesh,
        scratch_types=scratch,
        compiler_params=pltpu.CompilerParams(use_tc_tiling_on_sc=False),
    )
    def k(table_h, idx_h, w_h, out_h, acc_sh, ib, wb, zero_v, isem, wsem,
          zsem, *bufs):
        rows = bufs[:G]
        nb = 2 * G if bf16 else G
        rows16 = bufs[G:nb] if bf16 else rows
        gsems = bufs[nb:nb + G]
        ssem = bufs[nb + G]
        c = lax.axis_index("c")
        s = lax.axis_index("s")
        tbl = table_h.at[c] if feature_split else table_h
        base0 = 0 if feature_split else c * per_core

        zvec = jnp.zeros((16,), jnp.float32)

        @pl.loop(0, zr)
        def _(r):
            for jj in range(width // 16):
                zero_v[r, pl.ds(jj * 16, 16)] = zvec

        @pl.loop(s, n_zslabs, step=N_TILES)
        def _(i):
            pltpu.async_copy(zero_v, acc_sh.at[pl.ds(i * zr, zr)], zsem)

        @pl.loop(s, n_zslabs, step=N_TILES)
        def _(i):
            pltpu.make_async_copy(
                zero_v, acc_sh.at[pl.ds(s * zr, zr)], zsem).wait()
        plsc.subcore_barrier()

        def scale_rows(half, b):
            @pl.loop(0, ch // 16)
            def _(g):
                gbase = pl.multiple_of(g * 16, 16)
                wvec = wb[half, b, pl.ds(gbase, 16)]
                for l in range(16):
                    wl = wvec[l]
                    if bf16:
                        for q in range(width // 32):
                            v = rows16[b][gbase + l, pl.ds(q * 32, 32)]
                            a, d = plsc.unpack(
                                v, format=plsc.PackFormat.INTERLEAVED)
                            rows[b][gbase + l, pl.ds(q * 32, 16)] = a * wl
                            rows[b][gbase + l, pl.ds(q * 32 + 16, 16)] = d * wl
                    else:
                        for jj in range(width // 16):
                            sl = pl.ds(jj * 16, 16)
                            rows[b][gbase + l, sl] = rows[b][gbase + l, sl] * wl

        tile_c0 = base0 + s * cpt

        def do_group(grp, half, prefetch_grp):
            pltpu.make_async_copy(idx_h.at[pl.ds(tile_c0, G)], ib.at[half],
                                  isem).wait()
            pltpu.make_async_copy(w_h.at[pl.ds(tile_c0, G)], wb.at[half],
                                  wsem).wait()
            gathers = [
                pltpu.async_copy(tbl.at[ib.at[half, b, 0]], rows16[b],
                                 gsems[b])
                for b in range(G)
            ]
            if prefetch_grp is not None:
                @pl.when(prefetch_grp < n_groups)
                def _():
                    nc0 = tile_c0 + prefetch_grp * G
                    pltpu.async_copy(idx_h.at[pl.ds(nc0, G)],
                                     ib.at[1 - half], isem)
                    pltpu.async_copy(w_h.at[pl.ds(nc0, G)],
                                     wb.at[1 - half], wsem)
            scatters = []
            for b in range(G):
                gathers[b].wait()
                scale_rows(half, b)
                scatters.append(
                    pltpu.async_copy(rows[b], acc_sh.at[ib.at[half, b, 1]],
                                     ssem, add=True))
            for sc in scatters:
                sc.wait()

        pltpu.async_copy(idx_h.at[pl.ds(tile_c0, G)], ib.at[0], isem)
        pltpu.async_copy(w_h.at[pl.ds(tile_c0, G)], wb.at[0], wsem)

        @pl.loop(0, n_groups // 2)
        def _(m):
            do_group(2 * m, 0, 2 * m + 1)
            do_group(2 * m + 1, 1, 2 * m + 2)

        if n_groups % 2:
            do_group(n_groups - 1, 0, None)

        if leftover:
            @pl.when(s < leftover)
            def _():
                lc = base0 + N_TILES * cpt + s
                pltpu.sync_copy(idx_h.at[pl.ds(lc, 1)], ib.at[0, pl.ds(0, 1)])
                pltpu.sync_copy(w_h.at[pl.ds(lc, 1)], wb.at[0, pl.ds(0, 1)])
                pltpu.async_copy(tbl.at[ib.at[0, 0, 0]], rows16[0],
                                 gsems[0]).wait()
                scale_rows(0, 0)
                pltpu.sync_copy(rows[0], acc_sh.at[ib.at[0, 0, 1]], add=True)

        plsc.subcore_barrier()

        @pl.loop(s, n_oslabs, step=N_TILES)
        def _(i):
            r0 = pl.multiple_of(i * outr, outr)
            pltpu.async_copy(acc_sh.at[pl.ds(r0, outr)],
                             out_h.at[c, pl.ds(r0, outr)], zsem)

        @pl.loop(s, n_oslabs, step=N_TILES)
        def _(i):
            r0 = pl.multiple_of(s * outr, outr)
            pltpu.make_async_copy(acc_sh.at[pl.ds(r0, outr)],
                                  out_h.at[c, pl.ds(r0, outr)], zsem).wait()

    return k(table, packed, wchunk)


_ROWS = 1000


def _dot(a, b):
    return jnp.dot(a, b, preferred_element_type=jnp.float32)


def _ae_body(x_ref, e1w, e1b, zlw, zlb, d1w, d1b, xbw, xbb, cl,
             enc_o, z_o, xbar_o, q_o):
    xb = x_ref[...]
    e1 = jnp.maximum(_dot(xb, e1w[...]) + e1b[...], 0.0)
    z = _dot(e1, zlw[...]) + zlb[...]
    d1 = jnp.maximum(_dot(z, d1w[...]) + d1b[...], 0.0)
    xbar = _dot(d1, xbw[...]) + xbb[...]
    clv = cl[...]
    zz = jnp.sum(z * z, axis=1, keepdims=True)
    cc = jnp.sum(clv * clv, axis=1)[None, :]
    zc = lax.dot_general(z, clv, (((1,), (1,)), ((), ())),
                         preferred_element_type=jnp.float32)
    d2 = zz - 2.0 * zc + cc
    qq = 1.0 / (1.0 + d2 / V)
    q = qq / jnp.sum(qq, axis=1, keepdims=True)
    enc_o[...] = e1
    z_o[...] = z
    xbar_o[...] = xbar
    q_o[...] = q


def _dense_ae(x, enc1_w, enc1_b, zl_w, zl_b, dec1_w, dec1_b,
              xbar_w, xbar_b, cluster):
    n, n_in = x.shape
    n_e1 = enc1_w.shape[1]
    n_z = zl_w.shape[1]
    n_k = cluster.shape[0]
    grid = n // _ROWS
    full = lambda shp: pl.BlockSpec(shp, lambda i: (0,) * len(shp))
    row = lambda d: pl.BlockSpec((_ROWS, d), lambda i: (i, 0))
    return pl.pallas_call(
        _ae_body,
        grid=(grid,),
        in_specs=[
            row(n_in),
            full(enc1_w.shape), full((1, n_e1)),
            full(zl_w.shape), full((1, n_z)),
            full(dec1_w.shape), full((1, n_e1)),
            full(xbar_w.shape), full((1, n_in)),
            full(cluster.shape),
        ],
        out_specs=[row(n_e1), row(n_z), row(n_in), row(n_k)],
        out_shape=[
            jax.ShapeDtypeStruct((n, n_e1), jnp.float32),
            jax.ShapeDtypeStruct((n, n_z), jnp.float32),
            jax.ShapeDtypeStruct((n, n_in), jnp.float32),
            jax.ShapeDtypeStruct((n, n_k), jnp.float32),
        ],
    )(x, enc1_w, enc1_b.reshape(1, -1), zl_w, zl_b.reshape(1, -1),
      dec1_w, dec1_b.reshape(1, -1), xbar_w, xbar_b.reshape(1, -1), cluster)


def _gcn12_body(agg_ref, enc_ref, g1_ref, g4_ref, s2_o):
    h1 = jnp.maximum(_dot(agg_ref[0], g1_ref[0]) + _dot(agg_ref[1], g1_ref[1]),
                     0.0)
    mix = (1.0 - SIGMA) * h1 + SIGMA * enc_ref[...]
    s2 = _dot(mix, g4_ref[...])
    hz = s2.shape[1] // 2
    s2_o[0] = s2[:, :hz]
    s2_o[1] = s2[:, hz:]


def _gcn12(agg1, enc_h1, g1r, g4_w):
    n = enc_h1.shape[0]
    n_e1 = enc_h1.shape[1]
    n_z = g4_w.shape[1]
    hw = agg1.shape[2]
    grid = n // _ROWS
    return pl.pallas_call(
        _gcn12_body,
        grid=(grid,),
        in_specs=[
            pl.BlockSpec((N_CORES, _ROWS, hw), lambda i: (0, i, 0)),
            pl.BlockSpec((_ROWS, n_e1), lambda i: (i, 0)),
            pl.BlockSpec(g1r.shape, lambda i: (0, 0, 0)),
            pl.BlockSpec(g4_w.shape, lambda i: (0, 0)),
        ],
        out_specs=pl.BlockSpec((N_CORES, _ROWS, n_z // 2),
                               lambda i: (0, i, 0)),
        out_shape=jax.ShapeDtypeStruct((N_CORES, n, n_z // 2), jnp.float32),
    )(agg1, enc_h1, g1r, g4_w)


def _gcn3_body(agg_ref, z_ref, g5_ref, s3_o):
    h2 = jnp.maximum(jnp.concatenate([agg_ref[0], agg_ref[1]], axis=1), 0.0)
    mix = (1.0 - SIGMA) * h2 + SIGMA * z_ref[...]
    s3_o[...] = _dot(mix, g5_ref[...])


def _gcn3(agg2, z, g5_w):
    n, n_z = z.shape
    n_k = g5_w.shape[1]
    aw = agg2.shape[2]
    grid = n // _ROWS
    return pl.pallas_call(
        _gcn3_body,
        grid=(grid,),
        in_specs=[
            pl.BlockSpec((N_CORES, _ROWS, aw), lambda i: (0, i, 0)),
            pl.BlockSpec((_ROWS, n_z), lambda i: (i, 0)),
            pl.BlockSpec(g5_w.shape, lambda i: (0, 0)),
        ],
        out_specs=pl.BlockSpec((_ROWS, n_k), lambda i: (i, 0)),
        out_shape=jax.ShapeDtypeStruct((n, n_k), jnp.float32),
    )(agg2, z, g5_w)


def _softmax_body(agg_ref, pred_o):
    h3 = agg_ref[0] + agg_ref[1]
    m = jnp.max(h3, axis=1, keepdims=True)
    e = jnp.exp(h3 - m)
    pred_o[...] = e / jnp.sum(e, axis=1, keepdims=True)


def _softmax(agg3):
    n_k = agg3.shape[2]
    n = agg3.shape[1]
    grid = n // _ROWS
    return pl.pallas_call(
        _softmax_body,
        grid=(grid,),
        in_specs=[pl.BlockSpec((N_CORES, _ROWS, n_k), lambda i: (0, i, 0))],
        out_specs=pl.BlockSpec((_ROWS, n_k), lambda i: (i, 0)),
        out_shape=jax.ShapeDtypeStruct((n, n_k), jnp.float32),
    )(agg3)


def kernel(x, edge_index, edge_weight, enc1_w, enc1_b, zl_w, zl_b,
           dec1_w, dec1_b, xbar_w, xbar_b, g1_w, g4_w, g5_w, cluster):
    n, n_in = x.shape
    hw = n_in // N_CORES
    src = edge_index[0]
    dst = edge_index[1]
    def pack_idx(ch):
        e = src.shape[0]
        return jnp.stack([src.reshape(e // ch, ch),
                          dst.reshape(e // ch, ch)], axis=1)

    pk64 = pack_idx(64)
    pk128 = pack_idx(128)
    w64 = edge_weight.reshape(-1, 64)
    w128 = edge_weight.reshape(-1, 128)

    xs = jnp.stack([x[:, :hw], x[:, hw:]])
    g1r = g1_w.reshape(N_CORES, hw, g1_w.shape[1])

    agg1 = _spmm_sc(xs, pk128, w128, n, hw, feature_split=True, ch=128, G=2)

    enc_h1, z, x_bar, q = _dense_ae(
        x, enc1_w, enc1_b, zl_w, zl_b, dec1_w, dec1_b, xbar_w, xbar_b,
        cluster)

    s2 = _gcn12(agg1, enc_h1, g1r, g4_w)

    agg2 = _spmm_sc(s2, pk128, w128, n, s2.shape[2], feature_split=True,
                    ch=128, G=13)

    s3 = _gcn3(agg2, z, g5_w)

    agg3 = _spmm_sc(s3, pk128, w128, n, s3.shape[1], feature_split=False,
                    ch=128, G=13)

    predict = _softmax(agg3)

    return (x_bar, q, predict, z)


# speedup vs baseline: 1.2570x; 1.0034x over previous
import functools

import jax
import jax.numpy as jnp
from jax import lax
from jax.experimental import pallas as pl
from jax.experimental.pallas import tpu as pltpu
from jax.experimental.pallas import tpu_sc as plsc

N_CORES = 2
N_TILES = 16
CH = 128
SIGMA = 0.5
V = 1.0


def _spmm_sc(table, packed, wchunk, n_nodes, width, feature_split, ch, G):
    n_chunks = packed.shape[0]
    zr = 16
    n_zslabs = n_nodes // zr
    outr = 80
    n_oslabs = n_nodes // outr
    assert n_zslabs * zr == n_nodes and n_oslabs * outr == n_nodes

    per_core = n_chunks if feature_split else n_chunks // N_CORES
    cpt = per_core // N_TILES
    leftover = per_core - cpt * N_TILES
    assert cpt % G == 0
    n_groups = cpt // G

    mesh = plsc.VectorSubcoreMesh(core_axis_name="c", subcore_axis_name="s",
                                  num_cores=N_CORES, num_subcores=N_TILES)

    bf16 = table.dtype == jnp.bfloat16

    scratch = [
        pltpu.VMEM_SHARED((n_nodes, width), jnp.float32),
        pltpu.VMEM((2, G, 2, ch), jnp.int32),
        pltpu.VMEM((2, G, ch), jnp.float32),
        pltpu.VMEM((zr, width), jnp.float32),
        pltpu.SemaphoreType.DMA,
        pltpu.SemaphoreType.DMA,
        pltpu.SemaphoreType.DMA,
    ]
    scratch += [pltpu.VMEM((ch, width), jnp.float32) for _ in range(G)]
    if bf16:
        scratch += [pltpu.VMEM((ch, width), jnp.bfloat16) for _ in range(G)]
    scratch += [pltpu.SemaphoreType.DMA for _ in range(G)]
    scratch += [pltpu.SemaphoreType.DMA]

    @functools.partial(
        pl.kernel,
        out_type=jax.ShapeDtypeStruct((N_CORES, n_nodes, width), jnp.float32),
        mesh=mesh,
        scratch_types=scratch,
        compiler_params=pltpu.CompilerParams(use_tc_tiling_on_sc=False),
    )
    def k(table_h, idx_h, w_h, out_h, acc_sh, ib, wb, zero_v, isem, wsem,
          zsem, *bufs):
        rows = bufs[:G]
        nb = 2 * G if bf16 else G
        rows16 = bufs[G:nb] if bf16 else rows
        gsems = bufs[nb:nb + G]
        ssem = bufs[nb + G]
        c = lax.axis_index("c")
        s = lax.axis_index("s")
        tbl = table_h.at[c] if feature_split else table_h
        base0 = 0 if feature_split else c * per_core

        zvec = jnp.zeros((16,), jnp.float32)

        @pl.loop(0, zr)
        def _(r):
            for jj in range(width // 16):
                zero_v[r, pl.ds(jj * 16, 16)] = zvec

        @pl.loop(s, n_zslabs, step=N_TILES)
        def _(i):
            pltpu.async_copy(zero_v, acc_sh.at[pl.ds(i * zr, zr)], zsem)

        @pl.loop(s, n_zslabs, step=N_TILES)
        def _(i):
            pltpu.make_async_copy(
                zero_v, acc_sh.at[pl.ds(s * zr, zr)], zsem).wait()
        plsc.subcore_barrier()

        def scale_rows(half, b):
            @pl.loop(0, ch // 16)
            def _(g):
                gbase = pl.multiple_of(g * 16, 16)
                wvec = wb[half, b, pl.ds(gbase, 16)]
                for l in range(16):
                    wl = wvec[l]
                    if bf16:
                        for q in range(width // 32):
                            v = rows16[b][gbase + l, pl.ds(q * 32, 32)]
                            a, d = plsc.unpack(
                                v, format=plsc.PackFormat.INTERLEAVED)
                            rows[b][gbase + l, pl.ds(q * 32, 16)] = a * wl
                            rows[b][gbase + l, pl.ds(q * 32 + 16, 16)] = d * wl
                    else:
                        for jj in range(width // 16):
                            sl = pl.ds(jj * 16, 16)
                            rows[b][gbase + l, sl] = rows[b][gbase + l, sl] * wl

        tile_c0 = base0 + s * cpt

        def do_group(grp, half, prefetch_grp):
            pltpu.make_async_copy(idx_h.at[pl.ds(tile_c0, G)], ib.at[half],
                                  isem).wait()
            pltpu.make_async_copy(w_h.at[pl.ds(tile_c0, G)], wb.at[half],
                                  wsem).wait()
            gathers = [
                pltpu.async_copy(tbl.at[ib.at[half, b, 0]], rows16[b],
                                 gsems[b])
                for b in range(G)
            ]
            if prefetch_grp is not None:
                @pl.when(prefetch_grp < n_groups)
                def _():
                    nc0 = tile_c0 + prefetch_grp * G
                    pltpu.async_copy(idx_h.at[pl.ds(nc0, G)],
                                     ib.at[1 - half], isem)
                    pltpu.async_copy(w_h.at[pl.ds(nc0, G)],
                                     wb.at[1 - half], wsem)
            scatters = []
            for b in range(G):
                gathers[b].wait()
                scale_rows(half, b)
                scatters.append(
                    pltpu.async_copy(rows[b], acc_sh.at[ib.at[half, b, 1]],
                                     ssem, add=True))
            for sc in scatters:
                sc.wait()

        pltpu.async_copy(idx_h.at[pl.ds(tile_c0, G)], ib.at[0], isem)
        pltpu.async_copy(w_h.at[pl.ds(tile_c0, G)], wb.at[0], wsem)

        @pl.loop(0, n_groups // 2)
        def _(m):
            do_group(2 * m, 0, 2 * m + 1)
            do_group(2 * m + 1, 1, 2 * m + 2)

        if n_groups % 2:
            do_group(n_groups - 1, 0, None)

        if leftover:
            @pl.when(s < leftover)
            def _():
                lc = base0 + N_TILES * cpt + s
                pltpu.sync_copy(idx_h.at[pl.ds(lc, 1)], ib.at[0, pl.ds(0, 1)])
                pltpu.sync_copy(w_h.at[pl.ds(lc, 1)], wb.at[0, pl.ds(0, 1)])
                pltpu.async_copy(tbl.at[ib.at[0, 0, 0]], rows16[0],
                                 gsems[0]).wait()
                scale_rows(0, 0)
                pltpu.sync_copy(rows[0], acc_sh.at[ib.at[0, 0, 1]], add=True)

        plsc.subcore_barrier()

        @pl.loop(s, n_oslabs, step=N_TILES)
        def _(i):
            r0 = pl.multiple_of(i * outr, outr)
            pltpu.async_copy(acc_sh.at[pl.ds(r0, outr)],
                             out_h.at[c, pl.ds(r0, outr)], zsem)

        @pl.loop(s, n_oslabs, step=N_TILES)
        def _(i):
            r0 = pl.multiple_of(s * outr, outr)
            pltpu.make_async_copy(acc_sh.at[pl.ds(r0, outr)],
                                  out_h.at[c, pl.ds(r0, outr)], zsem).wait()

    return k(table, packed, wchunk)


_ROWS = 1000


def _dot(a, b):
    return jnp.dot(a, b, preferred_element_type=jnp.float32)


def _ae_body(x_ref, e1w, e1b, zlw, zlb, d1w, d1b, xbw, xbb, cl,
             enc_o, z_o, xbar_o, q_o):
    xb = x_ref[...]
    e1 = jnp.maximum(_dot(xb, e1w[...]) + e1b[...], 0.0)
    z = _dot(e1, zlw[...]) + zlb[...]
    d1 = jnp.maximum(_dot(z, d1w[...]) + d1b[...], 0.0)
    xbar = _dot(d1, xbw[...]) + xbb[...]
    clv = cl[...]
    zz = jnp.sum(z * z, axis=1, keepdims=True)
    cc = jnp.sum(clv * clv, axis=1)[None, :]
    zc = lax.dot_general(z, clv, (((1,), (1,)), ((), ())),
                         preferred_element_type=jnp.float32)
    d2 = zz - 2.0 * zc + cc
    qq = 1.0 / (1.0 + d2 / V)
    q = qq / jnp.sum(qq, axis=1, keepdims=True)
    enc_o[...] = e1
    z_o[...] = z
    xbar_o[...] = xbar
    q_o[...] = q


def _dense_ae(x, enc1_w, enc1_b, zl_w, zl_b, dec1_w, dec1_b,
              xbar_w, xbar_b, cluster):
    n, n_in = x.shape
    n_e1 = enc1_w.shape[1]
    n_z = zl_w.shape[1]
    n_k = cluster.shape[0]
    grid = n // _ROWS
    full = lambda shp: pl.BlockSpec(shp, lambda i: (0,) * len(shp))
    row = lambda d: pl.BlockSpec((_ROWS, d), lambda i: (i, 0))
    return pl.pallas_call(
        _ae_body,
        grid=(grid,),
        in_specs=[
            row(n_in),
            full(enc1_w.shape), full((1, n_e1)),
            full(zl_w.shape), full((1, n_z)),
            full(dec1_w.shape), full((1, n_e1)),
            full(xbar_w.shape), full((1, n_in)),
            full(cluster.shape),
        ],
        out_specs=[row(n_e1), row(n_z), row(n_in), row(n_k)],
        out_shape=[
            jax.ShapeDtypeStruct((n, n_e1), jnp.float32),
            jax.ShapeDtypeStruct((n, n_z), jnp.float32),
            jax.ShapeDtypeStruct((n, n_in), jnp.float32),
            jax.ShapeDtypeStruct((n, n_k), jnp.float32),
        ],
    )(x, enc1_w, enc1_b.reshape(1, -1), zl_w, zl_b.reshape(1, -1),
      dec1_w, dec1_b.reshape(1, -1), xbar_w, xbar_b.reshape(1, -1), cluster)


def _bdot(a, b):
    return jnp.dot(a.astype(jnp.bfloat16), b.astype(jnp.bfloat16),
                   preferred_element_type=jnp.float32)


def _gcn12_body(agg_ref, enc_ref, g1_ref, g4_ref, s2_o):
    h1 = jnp.maximum(
        _bdot(agg_ref[0], g1_ref[0]) + _bdot(agg_ref[1], g1_ref[1]), 0.0)
    mix = (1.0 - SIGMA) * h1 + SIGMA * enc_ref[...]
    s2 = _bdot(mix, g4_ref[...])
    hz = s2.shape[1] // 2
    s2_o[0] = s2[:, :hz]
    s2_o[1] = s2[:, hz:]


def _gcn12(agg1, enc_h1, g1r, g4_w):
    n = enc_h1.shape[0]
    n_e1 = enc_h1.shape[1]
    n_z = g4_w.shape[1]
    hw = agg1.shape[2]
    grid = n // _ROWS
    return pl.pallas_call(
        _gcn12_body,
        grid=(grid,),
        in_specs=[
            pl.BlockSpec((N_CORES, _ROWS, hw), lambda i: (0, i, 0)),
            pl.BlockSpec((_ROWS, n_e1), lambda i: (i, 0)),
            pl.BlockSpec(g1r.shape, lambda i: (0, 0, 0)),
            pl.BlockSpec(g4_w.shape, lambda i: (0, 0)),
        ],
        out_specs=pl.BlockSpec((N_CORES, _ROWS, n_z // 2),
                               lambda i: (0, i, 0)),
        out_shape=jax.ShapeDtypeStruct((N_CORES, n, n_z // 2), jnp.float32),
    )(agg1, enc_h1, g1r, g4_w)


def _gcn3_body(agg_ref, z_ref, g5_ref, s3_o):
    h2 = jnp.maximum(jnp.concatenate([agg_ref[0], agg_ref[1]], axis=1), 0.0)
    mix = (1.0 - SIGMA) * h2 + SIGMA * z_ref[...]
    s3_o[...] = _dot(mix, g5_ref[...])


def _gcn3(agg2, z, g5_w):
    n, n_z = z.shape
    n_k = g5_w.shape[1]
    aw = agg2.shape[2]
    grid = n // _ROWS
    return pl.pallas_call(
        _gcn3_body,
        grid=(grid,),
        in_specs=[
            pl.BlockSpec((N_CORES, _ROWS, aw), lambda i: (0, i, 0)),
            pl.BlockSpec((_ROWS, n_z), lambda i: (i, 0)),
            pl.BlockSpec(g5_w.shape, lambda i: (0, 0)),
        ],
        out_specs=pl.BlockSpec((_ROWS, n_k), lambda i: (i, 0)),
        out_shape=jax.ShapeDtypeStruct((n, n_k), jnp.float32),
    )(agg2, z, g5_w)


def _softmax_body(agg_ref, pred_o):
    h3 = agg_ref[0] + agg_ref[1]
    m = jnp.max(h3, axis=1, keepdims=True)
    e = jnp.exp(h3 - m)
    pred_o[...] = e / jnp.sum(e, axis=1, keepdims=True)


def _softmax(agg3):
    n_k = agg3.shape[2]
    n = agg3.shape[1]
    grid = n // _ROWS
    return pl.pallas_call(
        _softmax_body,
        grid=(grid,),
        in_specs=[pl.BlockSpec((N_CORES, _ROWS, n_k), lambda i: (0, i, 0))],
        out_specs=pl.BlockSpec((_ROWS, n_k), lambda i: (i, 0)),
        out_shape=jax.ShapeDtypeStruct((n, n_k), jnp.float32),
    )(agg3)


def kernel(x, edge_index, edge_weight, enc1_w, enc1_b, zl_w, zl_b,
           dec1_w, dec1_b, xbar_w, xbar_b, g1_w, g4_w, g5_w, cluster):
    n, n_in = x.shape
    hw = n_in // N_CORES
    src = edge_index[0]
    dst = edge_index[1]
    pk128 = jnp.transpose(edge_index.reshape(2, -1, 128), (1, 0, 2))
    w128 = edge_weight.reshape(-1, 128)

    xs = jnp.stack([x[:, :hw], x[:, hw:]])
    g1r = g1_w.reshape(N_CORES, hw, g1_w.shape[1])

    agg1 = _spmm_sc(xs, pk128, w128, n, hw, feature_split=True, ch=128, G=2)

    enc_h1, z, x_bar, q = _dense_ae(
        x, enc1_w, enc1_b, zl_w, zl_b, dec1_w, dec1_b, xbar_w, xbar_b,
        cluster)

    s2 = _gcn12(agg1, enc_h1, g1r, g4_w)

    agg2 = _spmm_sc(s2, pk128, w128, n, s2.shape[2], feature_split=True,
                    ch=128, G=13)

    s3 = _gcn3(agg2, z, g5_w)

    agg3 = _spmm_sc(s3, pk128, w128, n, s3.shape[1], feature_split=False,
                    ch=128, G=13)

    predict = _softmax(agg3)

    return (x_bar, q, predict, z)


# speedup vs baseline: 1.2947x; 1.0300x over previous
import functools

import jax
import jax.numpy as jnp
from jax import lax
from jax.experimental import pallas as pl
from jax.experimental.pallas import tpu as pltpu
from jax.experimental.pallas import tpu_sc as plsc

N_CORES = 2
N_TILES = 16
CH = 128
SIGMA = 0.5
V = 1.0


def _spmm_sc(table, packed, wchunk, n_nodes, width, feature_split, ch, G):
    n_chunks = packed.shape[0]
    zr = 16
    n_zslabs = n_nodes // zr
    outr = 80
    n_oslabs = n_nodes // outr
    assert n_zslabs * zr == n_nodes and n_oslabs * outr == n_nodes

    per_core = n_chunks if feature_split else n_chunks // N_CORES
    cpt = per_core // N_TILES
    leftover = per_core - cpt * N_TILES
    assert cpt % G == 0
    n_groups = cpt // G

    mesh = plsc.VectorSubcoreMesh(core_axis_name="c", subcore_axis_name="s",
                                  num_cores=N_CORES, num_subcores=N_TILES)

    bf16 = table.dtype == jnp.bfloat16

    scratch = [
        pltpu.VMEM_SHARED((n_nodes, width), jnp.float32),
        pltpu.VMEM((2, G, 2, ch), jnp.int32),
        pltpu.VMEM((2, G, ch), jnp.float32),
        pltpu.VMEM((zr, width), jnp.float32),
        pltpu.SemaphoreType.DMA,
        pltpu.SemaphoreType.DMA,
        pltpu.SemaphoreType.DMA,
    ]
    scratch += [pltpu.VMEM((ch, width), jnp.float32) for _ in range(G)]
    if bf16:
        scratch += [pltpu.VMEM((ch, width), jnp.bfloat16) for _ in range(G)]
    scratch += [pltpu.SemaphoreType.DMA for _ in range(G)]
    scratch += [pltpu.SemaphoreType.DMA]

    @functools.partial(
        pl.kernel,
        out_type=jax.ShapeDtypeStruct((N_CORES, n_nodes, width), jnp.float32),
        mesh=mesh,
        scratch_types=scratch,
        compiler_params=pltpu.CompilerParams(use_tc_tiling_on_sc=False),
    )
    def k(table_h, idx_h, w_h, out_h, acc_sh, ib, wb, zero_v, isem, wsem,
          zsem, *bufs):
        rows = bufs[:G]
        nb = 2 * G if bf16 else G
        rows16 = bufs[G:nb] if bf16 else rows
        gsems = bufs[nb:nb + G]
        ssem = bufs[nb + G]
        c = lax.axis_index("c")
        s = lax.axis_index("s")
        tbl = table_h.at[c] if feature_split else table_h
        base0 = 0 if feature_split else c * per_core

        tile_c0 = base0 + s * cpt
        pltpu.async_copy(idx_h.at[pl.ds(tile_c0, G)], ib.at[0], isem)
        pltpu.async_copy(w_h.at[pl.ds(tile_c0, G)], wb.at[0], wsem)

        zvec = jnp.zeros((16,), jnp.float32)

        @pl.loop(0, zr)
        def _(r):
            for jj in range(width // 16):
                zero_v[r, pl.ds(jj * 16, 16)] = zvec

        @pl.loop(s, n_zslabs, step=N_TILES)
        def _(i):
            pltpu.async_copy(zero_v, acc_sh.at[pl.ds(i * zr, zr)], zsem)

        @pl.loop(s, n_zslabs, step=N_TILES)
        def _(i):
            pltpu.make_async_copy(
                zero_v, acc_sh.at[pl.ds(s * zr, zr)], zsem).wait()
        plsc.subcore_barrier()

        def scale_rows(half, b):
            @pl.loop(0, ch // 16)
            def _(g):
                gbase = pl.multiple_of(g * 16, 16)
                wvec = wb[half, b, pl.ds(gbase, 16)]
                for l in range(16):
                    wl = wvec[l]
                    if bf16:
                        for q in range(width // 32):
                            v = rows16[b][gbase + l, pl.ds(q * 32, 32)]
                            a, d = plsc.unpack(
                                v, format=plsc.PackFormat.INTERLEAVED)
                            rows[b][gbase + l, pl.ds(q * 32, 16)] = a * wl
                            rows[b][gbase + l, pl.ds(q * 32 + 16, 16)] = d * wl
                    else:
                        for jj in range(width // 16):
                            sl = pl.ds(jj * 16, 16)
                            rows[b][gbase + l, sl] = rows[b][gbase + l, sl] * wl

        def do_group(grp, half, prefetch_grp):
            pltpu.make_async_copy(idx_h.at[pl.ds(tile_c0, G)], ib.at[half],
                                  isem).wait()
            pltpu.make_async_copy(w_h.at[pl.ds(tile_c0, G)], wb.at[half],
                                  wsem).wait()
            gathers = [
                pltpu.async_copy(tbl.at[ib.at[half, b, 0]], rows16[b],
                                 gsems[b])
                for b in range(G)
            ]
            if prefetch_grp is not None:
                @pl.when(prefetch_grp < n_groups)
                def _():
                    nc0 = tile_c0 + prefetch_grp * G
                    pltpu.async_copy(idx_h.at[pl.ds(nc0, G)],
                                     ib.at[1 - half], isem)
                    pltpu.async_copy(w_h.at[pl.ds(nc0, G)],
                                     wb.at[1 - half], wsem)
            scatters = []
            for b in range(G):
                gathers[b].wait()
                scale_rows(half, b)
                scatters.append(
                    pltpu.async_copy(rows[b], acc_sh.at[ib.at[half, b, 1]],
                                     ssem, add=True))
            for sc in scatters:
                sc.wait()

        @pl.loop(0, n_groups // 2)
        def _(m):
            do_group(2 * m, 0, 2 * m + 1)
            do_group(2 * m + 1, 1, 2 * m + 2)

        if n_groups % 2:
            do_group(n_groups - 1, 0, None)

        if leftover:
            @pl.when(s < leftover)
            def _():
                lc = base0 + N_TILES * cpt + s
                pltpu.sync_copy(idx_h.at[pl.ds(lc, 1)], ib.at[0, pl.ds(0, 1)])
                pltpu.sync_copy(w_h.at[pl.ds(lc, 1)], wb.at[0, pl.ds(0, 1)])
                pltpu.async_copy(tbl.at[ib.at[0, 0, 0]], rows16[0],
                                 gsems[0]).wait()
                scale_rows(0, 0)
                pltpu.sync_copy(rows[0], acc_sh.at[ib.at[0, 0, 1]], add=True)

        plsc.subcore_barrier()

        @pl.loop(s, n_oslabs, step=N_TILES)
        def _(i):
            r0 = pl.multiple_of(i * outr, outr)
            pltpu.async_copy(acc_sh.at[pl.ds(r0, outr)],
                             out_h.at[c, pl.ds(r0, outr)], zsem)

        @pl.loop(s, n_oslabs, step=N_TILES)
        def _(i):
            r0 = pl.multiple_of(s * outr, outr)
            pltpu.make_async_copy(acc_sh.at[pl.ds(r0, outr)],
                                  out_h.at[c, pl.ds(r0, outr)], zsem).wait()

    return k(table, packed, wchunk)


_ROWS = 2000


def _dot(a, b):
    return jnp.dot(a, b, preferred_element_type=jnp.float32)


def _ae_body(x_ref, e1w, e1b, zlw, zlb, d1w, d1b, xbw, xbb, cl,
             enc_o, z_o, xbar_o, q_o):
    xb = x_ref[...]
    e1 = jnp.maximum(_dot(xb, e1w[...]) + e1b[...], 0.0)
    z = _dot(e1, zlw[...]) + zlb[...]
    d1 = jnp.maximum(_dot(z, d1w[...]) + d1b[...], 0.0)
    xbar = _dot(d1, xbw[...]) + xbb[...]
    clv = cl[...]
    zz = jnp.sum(z * z, axis=1, keepdims=True)
    cc = jnp.sum(clv * clv, axis=1)[None, :]
    zc = lax.dot_general(z, clv, (((1,), (1,)), ((), ())),
                         preferred_element_type=jnp.float32)
    d2 = zz - 2.0 * zc + cc
    qq = 1.0 / (1.0 + d2 / V)
    q = qq / jnp.sum(qq, axis=1, keepdims=True)
    enc_o[...] = e1
    z_o[...] = z
    xbar_o[...] = xbar
    q_o[...] = q


def _dense_ae(x, enc1_w, enc1_b, zl_w, zl_b, dec1_w, dec1_b,
              xbar_w, xbar_b, cluster):
    n, n_in = x.shape
    n_e1 = enc1_w.shape[1]
    n_z = zl_w.shape[1]
    n_k = cluster.shape[0]
    grid = n // _ROWS
    full = lambda shp: pl.BlockSpec(shp, lambda i: (0,) * len(shp))
    row = lambda d: pl.BlockSpec((_ROWS, d), lambda i: (i, 0))
    return pl.pallas_call(
        _ae_body,
        grid=(grid,),
        in_specs=[
            row(n_in),
            full(enc1_w.shape), full((1, n_e1)),
            full(zl_w.shape), full((1, n_z)),
            full(dec1_w.shape), full((1, n_e1)),
            full(xbar_w.shape), full((1, n_in)),
            full(cluster.shape),
        ],
        out_specs=[row(n_e1), row(n_z), row(n_in), row(n_k)],
        out_shape=[
            jax.ShapeDtypeStruct((n, n_e1), jnp.float32),
            jax.ShapeDtypeStruct((n, n_z), jnp.float32),
            jax.ShapeDtypeStruct((n, n_in), jnp.float32),
            jax.ShapeDtypeStruct((n, n_k), jnp.float32),
        ],
    )(x, enc1_w, enc1_b.reshape(1, -1), zl_w, zl_b.reshape(1, -1),
      dec1_w, dec1_b.reshape(1, -1), xbar_w, xbar_b.reshape(1, -1), cluster)


def _bdot(a, b):
    return jnp.dot(a.astype(jnp.bfloat16), b.astype(jnp.bfloat16),
                   preferred_element_type=jnp.float32)


def _gcn12_body(agg_ref, enc_ref, g1_ref, g4_ref, s2_o):
    h1 = jnp.maximum(
        _bdot(agg_ref[0], g1_ref[0]) + _bdot(agg_ref[1], g1_ref[1]), 0.0)
    mix = (1.0 - SIGMA) * h1 + SIGMA * enc_ref[...]
    s2 = _bdot(mix, g4_ref[...])
    hz = s2.shape[1] // 2
    s2_o[0] = s2[:, :hz]
    s2_o[1] = s2[:, hz:]


def _gcn12(agg1, enc_h1, g1r, g4_w):
    n = enc_h1.shape[0]
    n_e1 = enc_h1.shape[1]
    n_z = g4_w.shape[1]
    hw = agg1.shape[2]
    grid = n // _ROWS
    return pl.pallas_call(
        _gcn12_body,
        grid=(grid,),
        in_specs=[
            pl.BlockSpec((N_CORES, _ROWS, hw), lambda i: (0, i, 0)),
            pl.BlockSpec((_ROWS, n_e1), lambda i: (i, 0)),
            pl.BlockSpec(g1r.shape, lambda i: (0, 0, 0)),
            pl.BlockSpec(g4_w.shape, lambda i: (0, 0)),
        ],
        out_specs=pl.BlockSpec((N_CORES, _ROWS, n_z // 2),
                               lambda i: (0, i, 0)),
        out_shape=jax.ShapeDtypeStruct((N_CORES, n, n_z // 2), jnp.float32),
    )(agg1, enc_h1, g1r, g4_w)


def _gcn3_body(agg_ref, z_ref, g5_ref, s3_o):
    h2 = jnp.maximum(jnp.concatenate([agg_ref[0], agg_ref[1]], axis=1), 0.0)
    mix = (1.0 - SIGMA) * h2 + SIGMA * z_ref[...]
    s3_o[...] = _dot(mix, g5_ref[...])


def _gcn3(agg2, z, g5_w):
    n, n_z = z.shape
    n_k = g5_w.shape[1]
    aw = agg2.shape[2]
    grid = n // _ROWS
    return pl.pallas_call(
        _gcn3_body,
        grid=(grid,),
        in_specs=[
            pl.BlockSpec((N_CORES, _ROWS, aw), lambda i: (0, i, 0)),
            pl.BlockSpec((_ROWS, n_z), lambda i: (i, 0)),
            pl.BlockSpec(g5_w.shape, lambda i: (0, 0)),
        ],
        out_specs=pl.BlockSpec((_ROWS, n_k), lambda i: (i, 0)),
        out_shape=jax.ShapeDtypeStruct((n, n_k), jnp.float32),
    )(agg2, z, g5_w)


def _softmax_body(agg_ref, pred_o):
    h3 = agg_ref[0] + agg_ref[1]
    m = jnp.max(h3, axis=1, keepdims=True)
    e = jnp.exp(h3 - m)
    pred_o[...] = e / jnp.sum(e, axis=1, keepdims=True)


def _softmax(agg3):
    n_k = agg3.shape[2]
    n = agg3.shape[1]
    grid = n // _ROWS
    return pl.pallas_call(
        _softmax_body,
        grid=(grid,),
        in_specs=[pl.BlockSpec((N_CORES, _ROWS, n_k), lambda i: (0, i, 0))],
        out_specs=pl.BlockSpec((_ROWS, n_k), lambda i: (i, 0)),
        out_shape=jax.ShapeDtypeStruct((n, n_k), jnp.float32),
    )(agg3)


def kernel(x, edge_index, edge_weight, enc1_w, enc1_b, zl_w, zl_b,
           dec1_w, dec1_b, xbar_w, xbar_b, g1_w, g4_w, g5_w, cluster):
    n, n_in = x.shape
    hw = n_in // N_CORES
    src = edge_index[0]
    dst = edge_index[1]
    pk128 = jnp.transpose(edge_index.reshape(2, -1, 128), (1, 0, 2))
    w128 = edge_weight.reshape(-1, 128)

    xs = jnp.stack([x[:, :hw], x[:, hw:]])
    g1r = g1_w.reshape(N_CORES, hw, g1_w.shape[1])

    agg1 = _spmm_sc(xs, pk128, w128, n, hw, feature_split=True, ch=128, G=2)

    enc_h1, z, x_bar, q = _dense_ae(
        x, enc1_w, enc1_b, zl_w, zl_b, dec1_w, dec1_b, xbar_w, xbar_b,
        cluster)

    s2 = _gcn12(agg1, enc_h1, g1r, g4_w)

    agg2 = _spmm_sc(s2, pk128, w128, n, s2.shape[2], feature_split=True,
                    ch=128, G=13)

    s3 = _gcn3(agg2, z, g5_w)

    agg3 = _spmm_sc(s3, pk128, w128, n, s3.shape[1], feature_split=False,
                    ch=128, G=13)

    predict = _softmax(agg3)

    return (x_bar, q, predict, z)
